# baseline (speedup 1.0000x reference)
import functools

import jax
import jax.numpy as jnp
from jax import lax
from jax.experimental import pallas as pl
from jax.experimental.pallas import tpu as pltpu
from jax.experimental.pallas import tpu_sc as plsc

N = 10000
E = 320000
D = 128

NC = 2
NS = 16
NW = NC * NS
L = 16

OWN = 320
NPAD = OWN * NW
G = 128
SUBCAP = 896
CAP = SUBCAP * L
CHUNK = 6400
NCH = E // CHUNK
DP = D // 2
NCC = D // (2 * L)

_mesh = plsc.VectorSubcoreMesh(core_axis_name="c", subcore_axis_name="s")
_sc_params = pltpu.CompilerParams(needs_layout_passes=False,
                                  use_tc_tiling_on_sc=False)


def _wid():
    return lax.axis_index("c") * NS + lax.axis_index("s")


def _zero_acc(acc_v):
    zero = jnp.zeros((L,), jnp.int32)

    def zrow(r, _):
        for c in range(NCC):
            acc_v[r, pl.ds(c * L, L)] = zero
        return 0

    lax.fori_loop(0, OWN, zrow, 0)


def _segmax_loop(m_hbm, srcl_v, offl_v, acc_v, rows_v, sems, ngroups):
    def start_gather(g, b):
        pltpu.make_async_copy(m_hbm.at[srcl_v.at[pl.ds(g * G, G)]],
                              rows_v.at[b], sems[b]).start()

    def wait_gather(b):
        pltpu.make_async_copy(m_hbm.at[srcl_v.at[pl.ds(0, G)]],
                              rows_v.at[b], sems[b]).wait()

    @pl.when(ngroups > 0)
    def _():
        start_gather(0, 0)

    def outer(gg, _):
        for b in range(2):
            g = gg * 2 + b

            @pl.when(g < ngroups)
            def _():
                @pl.when(g + 1 < ngroups)
                def _():
                    start_gather(g + 1, 1 - b)

                wait_gather(b)

                def blk_body(k, _):
                    ovec = offl_v[pl.ds(g * G + k * L, L)]
                    dsts = [ovec[j] for j in range(L)]
                    for j in range(L):
                        d = dsts[j]
                        e = k * L + j
                        rvals = [rows_v[b, e, pl.ds(c * L, L)]
                                 for c in range(NCC)]
                        avals = [acc_v[d, pl.ds(c * L, L)]
                                 for c in range(NCC)]
                        for c in range(NCC):
                            mx = jnp.maximum(
                                plsc.bitcast(avals[c], jnp.bfloat16),
                                plsc.bitcast(rvals[c], jnp.bfloat16))
                            acc_v[d, pl.ds(c * L, L)] = plsc.bitcast(
                                mx, jnp.int32)
                    return 0

                lax.fori_loop(0, G // L, blk_body, 0)
        return 0

    lax.fori_loop(0, (ngroups + 1) // 2, outer, 0)


@functools.partial(
    pl.kernel,
    out_type=(
        jax.ShapeDtypeStruct((NW, CAP), jnp.int32),
        jax.ShapeDtypeStruct((NW, CAP), jnp.int32),
        jax.ShapeDtypeStruct((NW, L), jnp.int32),
        jax.ShapeDtypeStruct((NPAD, DP), jnp.int32),
    ),
    mesh=_mesh,
    compiler_params=_sc_params,
    scratch_types=[
        pltpu.VMEM((2, CHUNK), jnp.int32),
        pltpu.VMEM((2, CHUNK), jnp.int32),
        pltpu.VMEM((CAP,), jnp.int32),
        pltpu.VMEM((CAP,), jnp.int32),
        pltpu.VMEM((CAP,), jnp.int32),
        pltpu.VMEM((CAP,), jnp.int32),
        pltpu.VMEM((L,), jnp.int32),
        pltpu.VMEM((OWN + 8, DP), jnp.int32),
        pltpu.VMEM((2, G, DP), jnp.int32),
        pltpu.SemaphoreType.DMA,
        pltpu.SemaphoreType.DMA,
        pltpu.SemaphoreType.DMA,
        pltpu.SemaphoreType.DMA,
    ],
)
def _compact_segmax(src_hbm, dst_hbm, m_hbm,
                    srcl_hbm, offl_hbm, cnt_hbm, agg_hbm,
                    srcc_v, dstc_v, srcs_v, offs_v, srcl_v, offl_v, cnt_v,
                    acc_v, rows_v, sem0, sem1, semg0, semg1):
    wid = _wid()
    lo = wid * OWN
    lane_base = lax.iota(jnp.int32, L) * SUBCAP
    sems = (sem0, sem1)

    def start_chunk(ci, b):
        base = ci * CHUNK
        pltpu.make_async_copy(src_hbm.at[pl.ds(base, CHUNK)],
                              srcc_v.at[b], sems[b]).start()
        pltpu.make_async_copy(dst_hbm.at[pl.ds(base, CHUNK)],
                              dstc_v.at[b], sems[b]).start()

    def wait_chunk(b):
        pltpu.make_async_copy(src_hbm.at[pl.ds(0, CHUNK)],
                              srcc_v.at[b], sems[b]).wait()
        pltpu.make_async_copy(dst_hbm.at[pl.ds(0, CHUNK)],
                              dstc_v.at[b], sems[b]).wait()

    start_chunk(0, 0)

    _zero_acc(acc_v)

    def outer(gg, ptrs):
        for b in range(2):
            ci = gg * 2 + b

            @pl.when(ci + 1 < NCH)
            def _():
                start_chunk(ci + 1, 1 - b)

            wait_chunk(b)

            UNROLL = 4

            def vec_body(i, p):
                base_i = i * (UNROLL * L)
                dvs = [dstc_v[b, pl.ds(base_i + u * L, L)]
                       for u in range(UNROLL)]
                svs = [srcc_v[b, pl.ds(base_i + u * L, L)]
                       for u in range(UNROLL)]
                os_ = [dv - lo for dv in dvs]
                ms = [o.astype(jnp.uint32) < jnp.uint32(OWN) for o in os_]
                for u in range(UNROLL):
                    pos = lane_base + p
                    plsc.store_scatter(srcs_v, [pos], svs[u], mask=ms[u])
                    plsc.store_scatter(offs_v, [pos], os_[u], mask=ms[u])
                    p = p + ms[u].astype(jnp.int32)
                return jnp.minimum(p, SUBCAP - L)

            ptrs = lax.fori_loop(0, CHUNK // (UNROLL * L), vec_body, ptrs)
        return ptrs

    ptrs = lax.fori_loop(0, NCH // 2, outer, jnp.zeros((L,), jnp.int32))

    off = jnp.int32(0)
    for l in range(L):
        c_l = ptrs[l]
        src_base = l * SUBCAP

        def cp(i, _, off=off, src_base=src_base):
            srcl_v[pl.ds(off + i * L, L)] = srcs_v[pl.ds(src_base + i * L, L)]
            offl_v[pl.ds(off + i * L, L)] = offs_v[pl.ds(src_base + i * L, L)]
            return 0

        lax.fori_loop(0, (c_l + (L - 1)) // L, cp, 0)
        off = off + c_l

    pad_s = jnp.zeros((L,), jnp.int32)
    pad_o = jnp.full((L,), OWN, jnp.int32)

    def pad_body(j, _):
        srcl_v[pl.ds(off + j * L, L)] = pad_s
        offl_v[pl.ds(off + j * L, L)] = pad_o
        return 0

    lax.fori_loop(0, G // L, pad_body, 0)

    cnt_v[...] = jnp.full((L,), off, jnp.int32)
    pltpu.sync_copy(srcl_v, srcl_hbm.at[wid])
    pltpu.sync_copy(offl_v, offl_hbm.at[wid])
    pltpu.sync_copy(cnt_v, cnt_hbm.at[wid])

    ngroups = (off + (G - 1)) // G
    _segmax_loop(m_hbm, srcl_v, offl_v, acc_v, rows_v, (semg0, semg1),
                 ngroups)
    pltpu.sync_copy(acc_v.at[pl.ds(0, OWN)], agg_hbm.at[pl.ds(wid * OWN, OWN)])


@functools.partial(
    pl.kernel,
    out_type=jax.ShapeDtypeStruct((NPAD, DP), jnp.int32),
    mesh=_mesh,
    compiler_params=_sc_params,
    scratch_types=[
        pltpu.VMEM((CAP,), jnp.int32),
        pltpu.VMEM((CAP,), jnp.int32),
        pltpu.VMEM((L,), jnp.int32),
        pltpu.VMEM((OWN + 8, DP), jnp.int32),
        pltpu.VMEM((2, G, DP), jnp.int32),
        pltpu.SemaphoreType.DMA,
        pltpu.SemaphoreType.DMA,
    ],
)
def _segmax(m_hbm, srcl_hbm, offl_hbm, cnt_hbm, agg_hbm,
            srcl_v, offl_v, cnt_v, acc_v, rows_v, sem0, sem1):
    wid = _wid()
    pltpu.sync_copy(srcl_hbm.at[wid], srcl_v)
    pltpu.sync_copy(offl_hbm.at[wid], offl_v)
    pltpu.sync_copy(cnt_hbm.at[wid], cnt_v)
    _zero_acc(acc_v)
    cnt = cnt_v[...][0]
    ngroups = (cnt + (G - 1)) // G
    _segmax_loop(m_hbm, srcl_v, offl_v, acc_v, rows_v, (sem0, sem1), ngroups)
    pltpu.sync_copy(acc_v.at[pl.ds(0, OWN)], agg_hbm.at[pl.ds(wid * OWN, OWN)])


BLK = 2000
NB = N // BLK


def _tc1_body(x_ref, wa_ref, ba_ref, wf_ref, bf_ref, m1_ref, p1_ref):
    x = x_ref[...]
    m1_ref[...] = jnp.maximum(
        jnp.dot(x, wa_ref[...], preferred_element_type=jnp.float32)
        + ba_ref[...], 0.0).astype(jnp.bfloat16)
    p1_ref[...] = (
        jnp.dot(x, wf_ref[...], preferred_element_type=jnp.float32)
        + bf_ref[...])


def _tc1(x, wa, ba, wf, bf):
    return pl.pallas_call(
        _tc1_body,
        grid=(NB,),
        in_specs=[
            pl.BlockSpec((BLK, D), lambda i: (i, 0)),
            pl.BlockSpec((D, D), lambda i: (0, 0)),
            pl.BlockSpec((1, D), lambda i: (0, 0)),
            pl.BlockSpec((D, D), lambda i: (0, 0)),
            pl.BlockSpec((1, D), lambda i: (0, 0)),
        ],
        out_specs=[
            pl.BlockSpec((BLK, D), lambda i: (i, 0)),
            pl.BlockSpec((BLK, D), lambda i: (i, 0)),
        ],
        out_shape=[
            jax.ShapeDtypeStruct((N, D), jnp.bfloat16),
            jax.ShapeDtypeStruct((N, D), jnp.float32),
        ],
    )(x, wa, ba.reshape(1, D), wf, bf.reshape(1, D))


def _tc2_body(p1_ref, a1_ref, wfb_ref, g_ref, be_ref, wa2_ref, ba2_ref,
              wf2_ref, bf2_ref, m2_ref, p2_ref, pre_s, st_s):
    ph = pl.program_id(0)
    i = pl.program_id(1)

    @pl.when(ph == 0)
    def _():
        pre = jnp.maximum(
            p1_ref[...]
            + jnp.dot(a1_ref[...].astype(jnp.float32), wfb_ref[...],
                      preferred_element_type=jnp.float32), 0.0)
        pre_s[pl.ds(i * BLK, BLK), :] = pre
        s0 = jnp.sum(pre, axis=0, keepdims=True)
        s1 = jnp.sum(pre * pre, axis=0, keepdims=True)

        @pl.when(i == 0)
        def _():
            st_s[0:1, :] = s0
            st_s[1:2, :] = s1

        @pl.when(i > 0)
        def _():
            st_s[0:1, :] = st_s[0:1, :] + s0
            st_s[1:2, :] = st_s[1:2, :] + s1

    @pl.when(ph == 1)
    def _():
        mean = st_s[0:1, :] / N
        var = st_s[1:2, :] / N - mean * mean
        inv = g_ref[...] * lax.rsqrt(var + 1e-5)
        x = (pre_s[pl.ds(i * BLK, BLK), :] - mean) * inv + be_ref[...]
        nrm = jnp.sqrt(jnp.sum(x * x, axis=1, keepdims=True))
        out1 = x / (nrm + 1e-6)
        m2_ref[...] = jnp.maximum(
            jnp.dot(out1, wa2_ref[...], preferred_element_type=jnp.float32)
            + ba2_ref[...], 0.0).astype(jnp.bfloat16)
        p2_ref[...] = (
            jnp.dot(out1, wf2_ref[...], preferred_element_type=jnp.float32)
            + bf2_ref[...])


def _tc2(p1, a1, wfb, gamma, beta, wa2, ba2, wf2, bf2):
    return pl.pallas_call(
        _tc2_body,
        grid=(2, NB),
        in_specs=[
            pl.BlockSpec((BLK, D), lambda p, i: (i, 0)),
            pl.BlockSpec((BLK, D), lambda p, i: (i, 0)),
            pl.BlockSpec((D, D), lambda p, i: (0, 0)),
            pl.BlockSpec((1, D), lambda p, i: (0, 0)),
            pl.BlockSpec((1, D), lambda p, i: (0, 0)),
            pl.BlockSpec((D, D), lambda p, i: (0, 0)),
            pl.BlockSpec((1, D), lambda p, i: (0, 0)),
            pl.BlockSpec((D, D), lambda p, i: (0, 0)),
            pl.BlockSpec((1, D), lambda p, i: (0, 0)),
        ],
        out_specs=[
            pl.BlockSpec((BLK, D), lambda p, i: (i, 0)),
            pl.BlockSpec((BLK, D), lambda p, i: (i, 0)),
        ],
        out_shape=[
            jax.ShapeDtypeStruct((N, D), jnp.bfloat16),
            jax.ShapeDtypeStruct((N, D), jnp.float32),
        ],
        scratch_shapes=[
            pltpu.VMEM((N, D), jnp.float32),
            pltpu.VMEM((8, D), jnp.float32),
        ],
    )(p1, a1, wfb, gamma.reshape(1, D), beta.reshape(1, D), wa2,
      ba2.reshape(1, D), wf2, bf2.reshape(1, D))


def _tc3_body(p2_ref, a2_ref, wfb_ref, o_ref):
    o_ref[...] = p2_ref[...] + jnp.dot(
        a2_ref[...].astype(jnp.float32), wfb_ref[...],
        preferred_element_type=jnp.float32)


def _tc3(p2, a2, wfb):
    return pl.pallas_call(
        _tc3_body,
        grid=(NB,),
        in_specs=[
            pl.BlockSpec((BLK, D), lambda i: (i, 0)),
            pl.BlockSpec((BLK, D), lambda i: (i, 0)),
            pl.BlockSpec((D, D), lambda i: (0, 0)),
        ],
        out_specs=pl.BlockSpec((BLK, D), lambda i: (i, 0)),
        out_shape=jax.ShapeDtypeStruct((N, D), jnp.float32),
    )(p2, a2, wfb)


def kernel(features, edge_index, W_agg1, b_agg1, W_fc1, b_fc1, gamma, beta,
           W_agg2, b_agg2, W_fc2, b_fc2):
    src = edge_index[0]
    dst = edge_index[1]

    def pack(m_bf16):
        return lax.bitcast_convert_type(
            m_bf16.reshape(N, DP, 2), jnp.int32)

    def unpack(agg_i32):
        return lax.bitcast_convert_type(
            agg_i32, jnp.bfloat16).reshape(NPAD, D)[:N]

    m1, partial1 = _tc1(features, W_agg1, b_agg1, W_fc1[:D], b_fc1)
    srcl, offl, cnts, agg1p = _compact_segmax(src, dst, pack(m1))
    agg1 = unpack(agg1p)
    m2, partial2 = _tc2(partial1, agg1, W_fc1[D:], gamma, beta,
                        W_agg2, b_agg2, W_fc2[:D], b_fc2)
    agg2 = unpack(_segmax(pack(m2), srcl, offl, cnts))
    return _tc3(partial2, agg2, W_fc2[D:])
